# baseline (speedup 1.0000x reference)
import functools

import jax
import jax.numpy as jnp
from jax import lax
from jax.experimental import pallas as pl
from jax.experimental.pallas import tpu as pltpu
from jax.experimental.pallas import tpu_sc as plsc

NC = 2
NS = 16
NW = NC * NS
CHUNK = 128


def _mesh():
    return plsc.VectorSubcoreMesh(core_axis_name="c", subcore_axis_name="s")


def _sc_degree(dst_idx, npad, nch):
    nr = (-(-npad // CHUNK) + NS * 8 - 1) // (NS * 8) * (NS * 8)
    nrt = nr // NS
    nsup = nch // SUP

    @functools.partial(
        pl.kernel,
        out_type=jax.ShapeDtypeStruct((NC, nr, CHUNK), jnp.float32),
        mesh=_mesh(),
        compiler_params=pltpu.CompilerParams(needs_layout_passes=False),
        scratch_types=[
            pltpu.VMEM((SUP, CHUNK), jnp.int32),
            pltpu.VMEM((nr, CHUNK), jnp.float32),
            pltpu.VMEM((nr,), jnp.int32),
            pltpu.VMEM_SHARED((nr, CHUNK), jnp.float32),
        ],
    )
    def k(dst_hbm, out_hbm, idx_d, hist, rowidx, acc):
        c = lax.axis_index("c")
        s = lax.axis_index("s")
        wid = c * NS + s
        zeros16 = jnp.zeros((16,), jnp.float32)
        ones16 = jnp.ones((16,), jnp.float32)

        def zero_body(i, carry):
            for v in range(CHUNK // 16):
                hist[i, pl.ds(v * 16, 16)] = zeros16
            return carry

        lax.fori_loop(0, nr, zero_body, None)
        for i in range(nr // 16):
            rowidx[pl.ds(i * 16, 16)] = lax.iota(jnp.int32, 16) + i * 16
        rows = pl.ds(s * nrt, nrt)
        pltpu.sync_copy(hist.at[rows], acc.at[rows])
        plsc.subcore_barrier()

        def sup_body(sup, carry):
            pltpu.sync_copy(dst_hbm.at[wid, sup], idx_d)
            for j in range(SUP):
                for v in range(CHUNK // 16):
                    vec = idx_d[j, pl.ds(v * 16, 16)]
                    plsc.addupdate_scatter(
                        hist,
                        [lax.shift_right_logical(vec, 7),
                         lax.bitwise_and(vec, 127)],
                        ones16,
                    )
            return carry

        lax.fori_loop(0, nsup, sup_body, None)
        pltpu.sync_copy(hist, acc.at[rowidx], add=True)
        plsc.subcore_barrier()
        pltpu.sync_copy(acc.at[rows], out_hbm.at[c, rows])

    return k(dst_idx)


SUP = 8


def _sc_aggregate(y, src_idx, dst_idx, zeros_big, npad, nch, d):
    rpt = npad // NS
    npair = nch // (2 * SUP)

    @functools.partial(
        pl.kernel,
        out_type=jax.ShapeDtypeStruct((NC, npad, d), jnp.float32),
        mesh=_mesh(),
        scratch_types=[
            pltpu.VMEM((SUP, CHUNK), jnp.int32),
            pltpu.VMEM((SUP, CHUNK), jnp.int32),
            pltpu.VMEM((SUP, CHUNK), jnp.int32),
            pltpu.VMEM((SUP, CHUNK), jnp.int32),
            pltpu.VMEM((2, CHUNK, d), jnp.float32),
            pltpu.SemaphoreType.DMA((2,)),
            pltpu.SemaphoreType.DMA((2,)),
            pltpu.VMEM_SHARED((npad, d), jnp.float32),
        ],
    )
    def k(y_hbm, src_hbm, dst_hbm, z_hbm, out_hbm,
          idx_s0, idx_d0, idx_s1, idx_d1, buf, gsem, ssem, acc):
        c = lax.axis_index("c")
        s = lax.axis_index("s")
        wid = c * NS + s
        rows = pl.ds(s * rpt, rpt)
        pltpu.sync_copy(z_hbm.at[rows], acc.at[rows])
        plsc.subcore_barrier()

        def chunk(t, k_off, isr, idr, j):
            slot = (k_off + j) % 2
            if k_off + j >= 2:
                pltpu.make_async_copy(
                    buf.at[slot], acc.at[idr.at[j]], ssem.at[slot]
                ).wait()
            else:
                @pl.when(t > 0)
                def _wait_prev():
                    pltpu.make_async_copy(
                        buf.at[slot], acc.at[idr.at[j]], ssem.at[slot]
                    ).wait()
            pltpu.async_copy(y_hbm.at[isr.at[j]], buf.at[slot],
                             gsem.at[slot])
            pltpu.make_async_copy(y_hbm.at[isr.at[j]], buf.at[slot],
                                  gsem.at[slot]).wait()
            pltpu.make_async_copy(
                buf.at[slot], acc.at[idr.at[j]], ssem.at[slot]
            ).start(add=True)

        def pair_body(t, carry):
            sup0 = 2 * t
            pltpu.sync_copy(src_hbm.at[wid, sup0], idx_s0)
            pltpu.sync_copy(dst_hbm.at[wid, sup0], idx_d0)
            for j in range(SUP):
                chunk(t, 0, idx_s0, idx_d0, j)
            pltpu.sync_copy(src_hbm.at[wid, sup0 + 1], idx_s1)
            pltpu.sync_copy(dst_hbm.at[wid, sup0 + 1], idx_d1)
            for j in range(SUP):
                chunk(t, SUP, idx_s1, idx_d1, j)
            return carry

        lax.fori_loop(0, npair, pair_body, None)
        for slot in range(2):
            pltpu.make_async_copy(
                buf.at[slot], acc.at[idx_d1.at[SUP - 2 + slot]],
                ssem.at[slot]
            ).wait()
        plsc.subcore_barrier()
        pltpu.sync_copy(acc.at[rows], out_hbm.at[c, rows])

    return k(y, src_idx, dst_idx, zeros_big)


def _tc_xw(x, w1, n, d):
    r = 1000
    g = n // r

    def body(x_ref, w_ref, o_ref):
        o_ref[...] = jnp.dot(x_ref[...], w_ref[...],
                             preferred_element_type=jnp.float32)

    return pl.pallas_call(
        body,
        grid=(g,),
        in_specs=[
            pl.BlockSpec((r, d), lambda i: (i, 0)),
            pl.BlockSpec((d, d), lambda i: (0, 0)),
        ],
        out_specs=pl.BlockSpec((r, d), lambda i: (i, 0)),
        out_shape=jax.ShapeDtypeStruct((n, d), jnp.float32),
    )(x, w1)


def _tc_scale(xw, degp, n, d):
    r = 1000
    g = n // r

    def body(xw_ref, dp_ref, y_ref, dinv_ref):
        deg = dp_ref[0] + dp_ref[1]
        dinv = lax.rsqrt(deg + 1.0)
        y_ref[...] = xw_ref[...] * dinv
        dinv_ref[...] = dinv

    return pl.pallas_call(
        body,
        grid=(g,),
        in_specs=[
            pl.BlockSpec((r, d), lambda i: (i, 0)),
            pl.BlockSpec((NC, r, 1), lambda i: (0, i, 0)),
        ],
        out_specs=[
            pl.BlockSpec((r, d), lambda i: (i, 0)),
            pl.BlockSpec((r, 1), lambda i: (i, 0)),
        ],
        out_shape=[
            jax.ShapeDtypeStruct((n, d), jnp.float32),
            jax.ShapeDtypeStruct((n, 1), jnp.float32),
        ],
    )(xw, degp)


def _tc_mid(p, y, dinv, b, w, n, d):
    r = 1000
    g = n // r

    def body(p_ref, y_ref, di_ref, b_ref, w_ref, o_ref):
        agg = p_ref[0] + p_ref[1] + y_ref[...]
        h = jnp.maximum(agg * di_ref[...] + b_ref[...], 0.0)
        o_ref[...] = (
            jnp.dot(h, w_ref[...], preferred_element_type=jnp.float32)
            * di_ref[...]
        )

    return pl.pallas_call(
        body,
        grid=(g,),
        in_specs=[
            pl.BlockSpec((NC, r, d), lambda i: (0, i, 0)),
            pl.BlockSpec((r, d), lambda i: (i, 0)),
            pl.BlockSpec((r, 1), lambda i: (i, 0)),
            pl.BlockSpec((1, d), lambda i: (0, 0)),
            pl.BlockSpec((d, d), lambda i: (0, 0)),
        ],
        out_specs=pl.BlockSpec((r, d), lambda i: (i, 0)),
        out_shape=jax.ShapeDtypeStruct((n, d), jnp.float32),
    )(p, y, dinv, b, w)


def _tc_head(q, y, dinv, b, wfc, bfc, n, d, dout):
    r = 1000
    g = n // r

    def body(q_ref, y_ref, di_ref, b_ref, w_ref, bo_ref, o_ref):
        agg = q_ref[0] + q_ref[1] + y_ref[...]
        h = jnp.maximum(agg * di_ref[...] + b_ref[...], 0.0)
        o_ref[...] = (
            jnp.dot(h, w_ref[...], preferred_element_type=jnp.float32)
            + bo_ref[...]
        )

    return pl.pallas_call(
        body,
        grid=(g,),
        in_specs=[
            pl.BlockSpec((NC, r, d), lambda i: (0, i, 0)),
            pl.BlockSpec((r, d), lambda i: (i, 0)),
            pl.BlockSpec((r, 1), lambda i: (i, 0)),
            pl.BlockSpec((1, d), lambda i: (0, 0)),
            pl.BlockSpec((d, dout), lambda i: (0, 0)),
            pl.BlockSpec((1, dout), lambda i: (0, 0)),
        ],
        out_specs=pl.BlockSpec((r, dout), lambda i: (i, 0)),
        out_shape=jax.ShapeDtypeStruct((n, dout), jnp.float32),
    )(q, y, dinv, b, wfc, bfc)


def kernel(x, edge_index, W1, b1, W2, b2, Wfc, bfc):
    n, d = x.shape
    e = edge_index.shape[1]
    dout = Wfc.shape[1]

    rpt = (-(-(n + 1) // NS) + 7) // 8 * 8
    npad = rpt * NS
    nch = (-(-e // (NW * CHUNK)) + 2 * SUP - 1) // (2 * SUP) * (2 * SUP)
    nsup = nch // SUP
    epad = NW * nch * CHUNK

    npads = epad - e
    pad_src = (jnp.arange(npads, dtype=jnp.int32) * 131) % n
    pad_dst = n + jnp.arange(npads, dtype=jnp.int32) % (npad - n)
    src = jnp.concatenate([edge_index[0].astype(jnp.int32), pad_src])
    dstf = jnp.concatenate([edge_index[1].astype(jnp.int32), pad_dst])
    src = src.reshape(epad // NW, NW).T.reshape(NW, nsup, SUP, CHUNK)
    dst = dstf.reshape(epad // NW, NW).T.reshape(NW, nsup, SUP, CHUNK)

    zeros_big = jnp.zeros((npad, d), jnp.float32)

    xw1 = _tc_xw(x, W1, n, d)
    degp = _sc_degree(dst, npad, nch)
    degp = degp.reshape(NC, -1)[:, :n, None]
    y1, dinv = _tc_scale(xw1, degp, n, d)
    p = _sc_aggregate(y1, src, dst, zeros_big, npad, nch, d)
    y2 = _tc_mid(p, y1, dinv, b1.reshape(1, d), W2, n, d)
    q = _sc_aggregate(y2, src, dst, zeros_big, npad, nch, d)
    return _tc_head(q, y2, dinv, b2.reshape(1, d), Wfc, bfc.reshape(1, dout),
                    n, d, dout)


# speedup vs baseline: 1.0296x; 1.0296x over previous
import functools

import jax
import jax.numpy as jnp
from jax import lax
from jax.experimental import pallas as pl
from jax.experimental.pallas import tpu as pltpu
from jax.experimental.pallas import tpu_sc as plsc

NC = 2
NS = 16
NW = NC * NS
CHUNK = 128


def _mesh():
    return plsc.VectorSubcoreMesh(core_axis_name="c", subcore_axis_name="s")


def _sc_degree(dst_idx, npad, nch):
    nr = (-(-npad // CHUNK) + NS * 8 - 1) // (NS * 8) * (NS * 8)
    nrt = nr // NS
    nsup = nch // SUP

    @functools.partial(
        pl.kernel,
        out_type=jax.ShapeDtypeStruct((NC, nr, CHUNK), jnp.float32),
        mesh=_mesh(),
        compiler_params=pltpu.CompilerParams(needs_layout_passes=False),
        scratch_types=[
            pltpu.VMEM((SUP, CHUNK), jnp.int32),
            pltpu.VMEM((nr, CHUNK), jnp.float32),
            pltpu.VMEM((nr,), jnp.int32),
            pltpu.VMEM_SHARED((nr, CHUNK), jnp.float32),
        ],
    )
    def k(dst_hbm, out_hbm, idx_d, hist, rowidx, acc):
        c = lax.axis_index("c")
        s = lax.axis_index("s")
        wid = c * NS + s
        zeros16 = jnp.zeros((16,), jnp.float32)
        ones16 = jnp.ones((16,), jnp.float32)

        def zero_body(i, carry):
            for v in range(CHUNK // 16):
                hist[i, pl.ds(v * 16, 16)] = zeros16
            return carry

        lax.fori_loop(0, nr, zero_body, None)
        for i in range(nr // 16):
            rowidx[pl.ds(i * 16, 16)] = lax.iota(jnp.int32, 16) + i * 16
        rows = pl.ds(s * nrt, nrt)
        pltpu.sync_copy(hist.at[rows], acc.at[rows])
        plsc.subcore_barrier()

        def sup_body(sup, carry):
            pltpu.sync_copy(dst_hbm.at[wid, sup], idx_d)
            for j in range(SUP):
                for v in range(CHUNK // 16):
                    vec = idx_d[j, pl.ds(v * 16, 16)]
                    plsc.addupdate_scatter(
                        hist,
                        [lax.shift_right_logical(vec, 7),
                         lax.bitwise_and(vec, 127)],
                        ones16,
                    )
            return carry

        lax.fori_loop(0, nsup, sup_body, None)
        pltpu.sync_copy(hist, acc.at[rowidx], add=True)
        plsc.subcore_barrier()
        pltpu.sync_copy(acc.at[rows], out_hbm.at[c, rows])

    return k(dst_idx)


SUP = 8


def _sc_aggregate(y, src_idx, dst_idx, zeros_big, npad, nch, d):
    rpt = npad // NS
    npair = nch // (2 * SUP)

    @functools.partial(
        pl.kernel,
        out_type=jax.ShapeDtypeStruct((NC, npad, d), jnp.float32),
        mesh=_mesh(),
        scratch_types=[
            pltpu.VMEM((SUP, CHUNK), jnp.int32),
            pltpu.VMEM((SUP, CHUNK), jnp.int32),
            pltpu.VMEM((SUP, CHUNK), jnp.int32),
            pltpu.VMEM((SUP, CHUNK), jnp.int32),
            pltpu.VMEM((2, CHUNK, d), jnp.float32),
            pltpu.SemaphoreType.DMA((2,)),
            pltpu.SemaphoreType.DMA((2,)),
            pltpu.VMEM_SHARED((npad, d), jnp.float32),
        ],
    )
    def k(y_hbm, src_hbm, dst_hbm, z_hbm, out_hbm,
          idx_s0, idx_d0, idx_s1, idx_d1, buf, gsem, ssem, acc):
        c = lax.axis_index("c")
        s = lax.axis_index("s")
        wid = c * NS + s
        rows = pl.ds(s * rpt, rpt)
        pltpu.sync_copy(z_hbm.at[rows], acc.at[rows])
        plsc.subcore_barrier()

        def chunk(t, k_off, isr, idr, j):
            slot = (k_off + j) % 2
            if k_off + j >= 2:
                pltpu.make_async_copy(
                    buf.at[slot], acc.at[idr.at[j]], ssem.at[slot]
                ).wait()
            else:
                @pl.when(t > 0)
                def _wait_prev():
                    pltpu.make_async_copy(
                        buf.at[slot], acc.at[idr.at[j]], ssem.at[slot]
                    ).wait()
            pltpu.async_copy(y_hbm.at[isr.at[j]], buf.at[slot],
                             gsem.at[slot])
            pltpu.make_async_copy(y_hbm.at[isr.at[j]], buf.at[slot],
                                  gsem.at[slot]).wait()
            pltpu.make_async_copy(
                buf.at[slot], acc.at[idr.at[j]], ssem.at[slot]
            ).start(add=True)

        def pair_body(t, carry):
            sup0 = 2 * t
            pltpu.sync_copy(src_hbm.at[wid, sup0], idx_s0)
            pltpu.sync_copy(dst_hbm.at[wid, sup0], idx_d0)
            for j in range(SUP):
                chunk(t, 0, idx_s0, idx_d0, j)
            pltpu.sync_copy(src_hbm.at[wid, sup0 + 1], idx_s1)
            pltpu.sync_copy(dst_hbm.at[wid, sup0 + 1], idx_d1)
            for j in range(SUP):
                chunk(t, SUP, idx_s1, idx_d1, j)
            return carry

        lax.fori_loop(0, npair, pair_body, None)
        for slot in range(2):
            pltpu.make_async_copy(
                buf.at[slot], acc.at[idx_d1.at[SUP - 2 + slot]],
                ssem.at[slot]
            ).wait()
        plsc.subcore_barrier()
        pltpu.sync_copy(acc.at[rows], out_hbm.at[c, rows])

    return k(y, src_idx, dst_idx, zeros_big)


def _tc_prep(x, w1, degp, n, d):
    r = 1000
    g = n // r

    def body(x_ref, w_ref, dp_ref, y_ref, dinv_ref):
        deg = dp_ref[0] + dp_ref[1]
        dinv = lax.rsqrt(deg + 1.0)
        xw = jnp.dot(x_ref[...], w_ref[...], preferred_element_type=jnp.float32)
        y_ref[...] = xw * dinv
        dinv_ref[...] = dinv

    return pl.pallas_call(
        body,
        grid=(g,),
        in_specs=[
            pl.BlockSpec((r, d), lambda i: (i, 0)),
            pl.BlockSpec((d, d), lambda i: (0, 0)),
            pl.BlockSpec((NC, r, 1), lambda i: (0, i, 0)),
        ],
        out_specs=[
            pl.BlockSpec((r, d), lambda i: (i, 0)),
            pl.BlockSpec((r, 1), lambda i: (i, 0)),
        ],
        out_shape=[
            jax.ShapeDtypeStruct((n, d), jnp.float32),
            jax.ShapeDtypeStruct((n, 1), jnp.float32),
        ],
    )(x, w1, degp)


def _tc_mid(p, y, dinv, b, w, n, d):
    r = 1000
    g = n // r

    def body(p_ref, y_ref, di_ref, b_ref, w_ref, o_ref):
        agg = p_ref[0] + p_ref[1] + y_ref[...]
        h = jnp.maximum(agg * di_ref[...] + b_ref[...], 0.0)
        o_ref[...] = (
            jnp.dot(h, w_ref[...], preferred_element_type=jnp.float32)
            * di_ref[...]
        )

    return pl.pallas_call(
        body,
        grid=(g,),
        in_specs=[
            pl.BlockSpec((NC, r, d), lambda i: (0, i, 0)),
            pl.BlockSpec((r, d), lambda i: (i, 0)),
            pl.BlockSpec((r, 1), lambda i: (i, 0)),
            pl.BlockSpec((1, d), lambda i: (0, 0)),
            pl.BlockSpec((d, d), lambda i: (0, 0)),
        ],
        out_specs=pl.BlockSpec((r, d), lambda i: (i, 0)),
        out_shape=jax.ShapeDtypeStruct((n, d), jnp.float32),
    )(p, y, dinv, b, w)


def _tc_head(q, y, dinv, b, wfc, bfc, n, d, dout):
    r = 1000
    g = n // r

    def body(q_ref, y_ref, di_ref, b_ref, w_ref, bo_ref, o_ref):
        agg = q_ref[0] + q_ref[1] + y_ref[...]
        h = jnp.maximum(agg * di_ref[...] + b_ref[...], 0.0)
        o_ref[...] = (
            jnp.dot(h, w_ref[...], preferred_element_type=jnp.float32)
            + bo_ref[...]
        )

    return pl.pallas_call(
        body,
        grid=(g,),
        in_specs=[
            pl.BlockSpec((NC, r, d), lambda i: (0, i, 0)),
            pl.BlockSpec((r, d), lambda i: (i, 0)),
            pl.BlockSpec((r, 1), lambda i: (i, 0)),
            pl.BlockSpec((1, d), lambda i: (0, 0)),
            pl.BlockSpec((d, dout), lambda i: (0, 0)),
            pl.BlockSpec((1, dout), lambda i: (0, 0)),
        ],
        out_specs=pl.BlockSpec((r, dout), lambda i: (i, 0)),
        out_shape=jax.ShapeDtypeStruct((n, dout), jnp.float32),
    )(q, y, dinv, b, wfc, bfc)


def kernel(x, edge_index, W1, b1, W2, b2, Wfc, bfc):
    n, d = x.shape
    e = edge_index.shape[1]
    dout = Wfc.shape[1]

    rpt = (-(-(n + 1) // NS) + 7) // 8 * 8
    npad = rpt * NS
    nch = (-(-e // (NW * CHUNK)) + 2 * SUP - 1) // (2 * SUP) * (2 * SUP)
    nsup = nch // SUP
    epad = NW * nch * CHUNK

    npads = epad - e
    pad_src = (jnp.arange(npads, dtype=jnp.int32) * 131) % n
    pad_dst = n + jnp.arange(npads, dtype=jnp.int32) % (npad - n)
    src = jnp.concatenate([edge_index[0].astype(jnp.int32), pad_src])
    dstf = jnp.concatenate([edge_index[1].astype(jnp.int32), pad_dst])
    src = src.reshape(NW, nsup, SUP, CHUNK)
    dst = dstf.reshape(NW, nsup, SUP, CHUNK)

    zeros_big = jnp.zeros((npad, d), jnp.float32)

    degp = _sc_degree(dst, npad, nch)
    degp = degp.reshape(NC, -1)[:, :n, None]
    y1, dinv = _tc_prep(x, W1, degp, n, d)
    p = _sc_aggregate(y1, src, dst, zeros_big, npad, nch, d)
    y2 = _tc_mid(p, y1, dinv, b1.reshape(1, d), W2, n, d)
    q = _sc_aggregate(y2, src, dst, zeros_big, npad, nch, d)
    return _tc_head(q, y2, dinv, b2.reshape(1, d), Wfc, bfc.reshape(1, dout),
                    n, d, dout)


# speedup vs baseline: 1.0422x; 1.0122x over previous
import functools

import jax
import jax.numpy as jnp
from jax import lax
from jax.experimental import pallas as pl
from jax.experimental.pallas import tpu as pltpu
from jax.experimental.pallas import tpu_sc as plsc

NC = 2
NS = 16
NW = NC * NS
CHUNK = 128


def _mesh():
    return plsc.VectorSubcoreMesh(core_axis_name="c", subcore_axis_name="s")


def _sc_degree(dst_idx, npad, nch):
    nr = (-(-npad // CHUNK) + NS * 8 - 1) // (NS * 8) * (NS * 8)
    nrt = nr // NS
    nsup = nch // SUP

    @functools.partial(
        pl.kernel,
        out_type=jax.ShapeDtypeStruct((NC, nr, CHUNK), jnp.float32),
        mesh=_mesh(),
        compiler_params=pltpu.CompilerParams(needs_layout_passes=False),
        scratch_types=[
            pltpu.VMEM((SUP, CHUNK), jnp.int32),
            pltpu.VMEM((nr, CHUNK), jnp.float32),
            pltpu.VMEM((nr,), jnp.int32),
            pltpu.VMEM_SHARED((nr, CHUNK), jnp.float32),
        ],
    )
    def k(dst_hbm, out_hbm, idx_d, hist, rowidx, acc):
        c = lax.axis_index("c")
        s = lax.axis_index("s")
        wid = c * NS + s
        zeros16 = jnp.zeros((16,), jnp.float32)
        ones16 = jnp.ones((16,), jnp.float32)

        def zero_body(i, carry):
            for v in range(CHUNK // 16):
                hist[i, pl.ds(v * 16, 16)] = zeros16
            return carry

        lax.fori_loop(0, nr, zero_body, None)
        for i in range(nr // 16):
            rowidx[pl.ds(i * 16, 16)] = lax.iota(jnp.int32, 16) + i * 16
        rows = pl.ds(s * nrt, nrt)
        pltpu.sync_copy(hist.at[rows], acc.at[rows])
        plsc.subcore_barrier()

        def sup_body(sup, carry):
            pltpu.sync_copy(dst_hbm.at[wid, sup], idx_d)
            for j in range(SUP):
                for v in range(CHUNK // 16):
                    vec = idx_d[j, pl.ds(v * 16, 16)]
                    plsc.addupdate_scatter(
                        hist,
                        [lax.shift_right_logical(vec, 7),
                         lax.bitwise_and(vec, 127)],
                        ones16,
                    )
            return carry

        lax.fori_loop(0, nsup, sup_body, None)
        pltpu.sync_copy(hist, acc.at[rowidx], add=True)
        plsc.subcore_barrier()
        pltpu.sync_copy(acc.at[rows], out_hbm.at[c, rows])

    return k(dst_idx)


SUP = 8


def _sc_aggregate(y, src_idx, dst_idx, zeros_big, npad, nch, d):
    rpt = npad // NS
    npair = nch // (2 * SUP)

    @functools.partial(
        pl.kernel,
        out_type=jax.ShapeDtypeStruct((NC, npad, d), jnp.float32),
        mesh=_mesh(),
        scratch_types=[
            pltpu.VMEM((SUP, CHUNK), jnp.int32),
            pltpu.VMEM((SUP, CHUNK), jnp.int32),
            pltpu.VMEM((SUP, CHUNK), jnp.int32),
            pltpu.VMEM((SUP, CHUNK), jnp.int32),
            pltpu.VMEM((2, CHUNK, d), jnp.float32),
            pltpu.SemaphoreType.DMA((2,)),
            pltpu.SemaphoreType.DMA((2,)),
            pltpu.VMEM_SHARED((npad, d), jnp.float32),
        ],
    )
    def k(y_hbm, src_hbm, dst_hbm, z_hbm, out_hbm,
          idx_s0, idx_d0, idx_s1, idx_d1, buf, gsem, ssem, acc):
        c = lax.axis_index("c")
        s = lax.axis_index("s")
        wid = c * NS + s
        rows = pl.ds(s * rpt, rpt)
        pltpu.sync_copy(z_hbm.at[rows], acc.at[rows])
        plsc.subcore_barrier()

        def chunk(t, k_off, isr, idr, j):
            slot = (k_off + j) % 2
            if k_off + j >= 2:
                pltpu.make_async_copy(
                    buf.at[slot], acc.at[idr.at[j]], ssem.at[slot]
                ).wait()
            else:
                @pl.when(t > 0)
                def _wait_prev():
                    pltpu.make_async_copy(
                        buf.at[slot], acc.at[idr.at[j]], ssem.at[slot]
                    ).wait()
            pltpu.async_copy(y_hbm.at[isr.at[j]], buf.at[slot],
                             gsem.at[slot])
            pltpu.make_async_copy(y_hbm.at[isr.at[j]], buf.at[slot],
                                  gsem.at[slot]).wait()
            pltpu.make_async_copy(
                buf.at[slot], acc.at[idr.at[j]], ssem.at[slot]
            ).start(add=True)

        def pair_body(t, carry):
            sup0 = 2 * t
            pltpu.sync_copy(src_hbm.at[wid, sup0], idx_s0)
            pltpu.sync_copy(dst_hbm.at[wid, sup0], idx_d0)
            for j in range(SUP):
                chunk(t, 0, idx_s0, idx_d0, j)
            pltpu.sync_copy(src_hbm.at[wid, sup0 + 1], idx_s1)
            pltpu.sync_copy(dst_hbm.at[wid, sup0 + 1], idx_d1)
            for j in range(SUP):
                chunk(t, SUP, idx_s1, idx_d1, j)
            return carry

        lax.fori_loop(0, npair, pair_body, None)
        for slot in range(2):
            pltpu.make_async_copy(
                buf.at[slot], acc.at[idx_d1.at[SUP - 2 + slot]],
                ssem.at[slot]
            ).wait()
        plsc.subcore_barrier()
        pltpu.sync_copy(acc.at[rows], out_hbm.at[c, rows])

    return k(y, src_idx, dst_idx, zeros_big)


def _tc_prep(x, w1, degp, n, d):
    r = 2000
    g = n // r

    def body(x_ref, w_ref, dp_ref, y_ref, dinv_ref):
        deg = dp_ref[0] + dp_ref[1]
        dinv = lax.rsqrt(deg + 1.0)
        xw = jnp.dot(x_ref[...], w_ref[...], preferred_element_type=jnp.float32)
        y_ref[...] = xw * dinv
        dinv_ref[...] = dinv

    return pl.pallas_call(
        body,
        grid=(g,),
        in_specs=[
            pl.BlockSpec((r, d), lambda i: (i, 0)),
            pl.BlockSpec((d, d), lambda i: (0, 0)),
            pl.BlockSpec((NC, r, 1), lambda i: (0, i, 0)),
        ],
        out_specs=[
            pl.BlockSpec((r, d), lambda i: (i, 0)),
            pl.BlockSpec((r, 1), lambda i: (i, 0)),
        ],
        out_shape=[
            jax.ShapeDtypeStruct((n, d), jnp.float32),
            jax.ShapeDtypeStruct((n, 1), jnp.float32),
        ],
    )(x, w1, degp)


def _tc_mid(p, y, dinv, b, w, n, d):
    r = 2000
    g = n // r

    def body(p_ref, y_ref, di_ref, b_ref, w_ref, o_ref):
        agg = p_ref[0] + p_ref[1] + y_ref[...]
        h = jnp.maximum(agg * di_ref[...] + b_ref[...], 0.0)
        o_ref[...] = (
            jnp.dot(h, w_ref[...], preferred_element_type=jnp.float32)
            * di_ref[...]
        )

    return pl.pallas_call(
        body,
        grid=(g,),
        in_specs=[
            pl.BlockSpec((NC, r, d), lambda i: (0, i, 0)),
            pl.BlockSpec((r, d), lambda i: (i, 0)),
            pl.BlockSpec((r, 1), lambda i: (i, 0)),
            pl.BlockSpec((1, d), lambda i: (0, 0)),
            pl.BlockSpec((d, d), lambda i: (0, 0)),
        ],
        out_specs=pl.BlockSpec((r, d), lambda i: (i, 0)),
        out_shape=jax.ShapeDtypeStruct((n, d), jnp.float32),
    )(p, y, dinv, b, w)


def _tc_head(q, y, dinv, b, wfc, bfc, n, d, dout):
    r = 2000
    g = n // r

    def body(q_ref, y_ref, di_ref, b_ref, w_ref, bo_ref, o_ref):
        agg = q_ref[0] + q_ref[1] + y_ref[...]
        h = jnp.maximum(agg * di_ref[...] + b_ref[...], 0.0)
        o_ref[...] = (
            jnp.dot(h, w_ref[...], preferred_element_type=jnp.float32)
            + bo_ref[...]
        )

    return pl.pallas_call(
        body,
        grid=(g,),
        in_specs=[
            pl.BlockSpec((NC, r, d), lambda i: (0, i, 0)),
            pl.BlockSpec((r, d), lambda i: (i, 0)),
            pl.BlockSpec((r, 1), lambda i: (i, 0)),
            pl.BlockSpec((1, d), lambda i: (0, 0)),
            pl.BlockSpec((d, dout), lambda i: (0, 0)),
            pl.BlockSpec((1, dout), lambda i: (0, 0)),
        ],
        out_specs=pl.BlockSpec((r, dout), lambda i: (i, 0)),
        out_shape=jax.ShapeDtypeStruct((n, dout), jnp.float32),
    )(q, y, dinv, b, wfc, bfc)


def kernel(x, edge_index, W1, b1, W2, b2, Wfc, bfc):
    n, d = x.shape
    e = edge_index.shape[1]
    dout = Wfc.shape[1]

    rpt = (-(-(n + 1) // NS) + 7) // 8 * 8
    npad = rpt * NS
    nch = (-(-e // (NW * CHUNK)) + 2 * SUP - 1) // (2 * SUP) * (2 * SUP)
    nsup = nch // SUP
    epad = NW * nch * CHUNK

    npads = epad - e
    pad_src = (jnp.arange(npads, dtype=jnp.int32) * 131) % n
    pad_dst = n + jnp.arange(npads, dtype=jnp.int32) % (npad - n)
    src = jnp.concatenate([edge_index[0].astype(jnp.int32), pad_src])
    dstf = jnp.concatenate([edge_index[1].astype(jnp.int32), pad_dst])
    src = src.reshape(NW, nsup, SUP, CHUNK)
    dst = dstf.reshape(NW, nsup, SUP, CHUNK)

    zeros_big = jnp.zeros((npad, d), jnp.float32)

    degp = _sc_degree(dst, npad, nch)
    degp = degp.reshape(NC, -1)[:, :n, None]
    y1, dinv = _tc_prep(x, W1, degp, n, d)
    p = _sc_aggregate(y1, src, dst, zeros_big, npad, nch, d)
    y2 = _tc_mid(p, y1, dinv, b1.reshape(1, d), W2, n, d)
    q = _sc_aggregate(y2, src, dst, zeros_big, npad, nch, d)
    return _tc_head(q, y2, dinv, b2.reshape(1, d), Wfc, bfc.reshape(1, dout),
                    n, d, dout)
